# baseline (speedup 1.0000x reference)
import jax
import jax.numpy as jnp
from jax.experimental import pallas as pl
from jax.experimental.pallas import tpu as pltpu

NUM_CLASSES = 1000
D = 256
B = 4096
KPAD = 1024
CH = 1024
NCH = B // CH


def _tc_body(lab_ref, x_hbm, cent_hbm, out_ref,
             xbuf, cvm, cbf_ref, sem_x, sem_c):
    cpx = []
    for c in range(NCH):
        cp = pltpu.make_async_copy(
            x_hbm.at[pl.ds(c * CH, CH), :], xbuf.at[c], sem_x.at[c])
        cp.start()
        cpx.append(cp)
    cpc = pltpu.make_async_copy(cent_hbm, cvm, sem_c)
    cpc.start()

    labs = lab_ref[0, 0, :]
    iota_k = jax.lax.broadcasted_iota(jnp.int32, (CH, KPAD), 1)

    cpc.wait()
    cb = cvm[...].astype(jnp.bfloat16)
    pad = jnp.zeros((KPAD - NUM_CLASSES, D), jnp.bfloat16)
    cbf_ref[...] = jnp.concatenate([cb, pad], axis=0)
    cbf = cbf_ref[...]

    total = jnp.zeros((), jnp.float32)
    for c in range(NCH):
        lc = jax.lax.slice_in_dim(labs, c * CH, (c + 1) * CH)
        onehot = (lc[:, None] == iota_k).astype(jnp.bfloat16)
        g = jnp.dot(onehot, cbf, preferred_element_type=jnp.float32)
        cpx[c].wait()
        d = xbuf[c] - g
        total = total + jnp.sum(d * d)

    out_ref[...] = (total * (1.0 / B)).reshape(1, 1)


def kernel(x, labels, centers):
    labels_i32 = labels.astype(jnp.int32)
    loss = pl.pallas_call(
        _tc_body,
        in_specs=[
            pl.BlockSpec((1, 1, B), lambda: (0, 0, 0)),
            pl.BlockSpec(memory_space=pl.ANY),
            pl.BlockSpec(memory_space=pl.ANY),
        ],
        out_specs=pl.BlockSpec((1, 1), lambda: (0, 0)),
        out_shape=jax.ShapeDtypeStruct((1, 1), jnp.float32),
        scratch_shapes=[
            pltpu.VMEM((NCH, CH, D), jnp.float32),
            pltpu.VMEM((NUM_CLASSES, D), jnp.float32),
            pltpu.VMEM((KPAD, D), jnp.bfloat16),
            pltpu.SemaphoreType.DMA((NCH,)),
            pltpu.SemaphoreType.DMA,
        ],
    )(labels_i32.reshape(1, 1, B), x, centers)
    return loss[0, 0]


# speedup vs baseline: 1.4276x; 1.4276x over previous
import jax
import jax.numpy as jnp
from jax.experimental import pallas as pl
from jax.experimental.pallas import tpu as pltpu

NUM_CLASSES = 1000
D = 256
B = 4096
KPAD = 1024
BB = 2048
NBLK = B // BB


def _tc_body(x_ref, lab_ref, cent_hbm, out_ref, cvm, cbf_ref, sem_c):
    i = pl.program_id(0)

    @pl.when(i == 0)
    def _start_cent():
        pltpu.make_async_copy(cent_hbm, cvm, sem_c).start()

    labs = lab_ref[0, 0, :]
    iota_k = jax.lax.broadcasted_iota(jnp.int32, (BB, KPAD), 1)
    onehot = (labs[:, None] == iota_k).astype(jnp.bfloat16)

    @pl.when(i == 0)
    def _prep():
        pltpu.make_async_copy(cent_hbm, cvm, sem_c).wait()
        cb = cvm[...].astype(jnp.bfloat16)
        pad = jnp.zeros((KPAD - NUM_CLASSES, D), jnp.bfloat16)
        cbf_ref[...] = jnp.concatenate([cb, pad], axis=0)

    g = jnp.dot(onehot, cbf_ref[...],
                preferred_element_type=jnp.float32)
    d = x_ref[...] - g
    part = jnp.sum(d * d).reshape(1, 1)

    @pl.when(i == 0)
    def _init():
        out_ref[...] = part

    @pl.when(i != 0)
    def _acc():
        out_ref[...] += part

    @pl.when(i == NBLK - 1)
    def _fin():
        out_ref[...] = out_ref[...] * (1.0 / B)


def kernel(x, labels, centers):
    labels_i32 = labels.astype(jnp.int32)
    loss = pl.pallas_call(
        _tc_body,
        grid=(NBLK,),
        in_specs=[
            pl.BlockSpec((BB, D), lambda i: (i, 0)),
            pl.BlockSpec((1, 1, BB), lambda i: (i, 0, 0)),
            pl.BlockSpec(memory_space=pl.ANY),
        ],
        out_specs=pl.BlockSpec((1, 1), lambda i: (0, 0)),
        out_shape=jax.ShapeDtypeStruct((1, 1), jnp.float32),
        scratch_shapes=[
            pltpu.VMEM((NUM_CLASSES, D), jnp.float32),
            pltpu.VMEM((KPAD, D), jnp.bfloat16),
            pltpu.SemaphoreType.DMA,
        ],
    )(x, labels_i32.reshape(NBLK, 1, BB), centers)
    return loss[0, 0]


# speedup vs baseline: 1.7204x; 1.2051x over previous
import jax
import jax.numpy as jnp
from jax.experimental import pallas as pl
from jax.experimental.pallas import tpu as pltpu

NUM_CLASSES = 1000
D = 256
B = 4096
KPAD = 1024
BB = 2048
NBLK = B // BB


def _tc_body(x_ref, lab_ref, cent_ref, out_ref, cbf_ref):
    i = pl.program_id(0)

    @pl.when(i == 0)
    def _prep():
        cb = cent_ref[...].astype(jnp.bfloat16)
        pad = jnp.zeros((KPAD - NUM_CLASSES, D), jnp.bfloat16)
        cbf_ref[...] = jnp.concatenate([cb, pad], axis=0)

    labs = lab_ref[0, 0, :]
    iota_k = jax.lax.broadcasted_iota(jnp.int32, (BB, KPAD), 1)
    onehot = (labs[:, None] == iota_k).astype(jnp.bfloat16)
    g = jnp.dot(onehot, cbf_ref[...],
                preferred_element_type=jnp.float32)
    d = x_ref[...] - g
    part = jnp.sum(d * d).reshape(1, 1)

    @pl.when(i == 0)
    def _init():
        out_ref[...] = part

    @pl.when(i != 0)
    def _acc():
        out_ref[...] += part

    @pl.when(i == NBLK - 1)
    def _fin():
        out_ref[...] = out_ref[...] * (1.0 / B)


def kernel(x, labels, centers):
    labels_i32 = labels.astype(jnp.int32)
    loss = pl.pallas_call(
        _tc_body,
        grid=(NBLK,),
        in_specs=[
            pl.BlockSpec((BB, D), lambda i: (i, 0)),
            pl.BlockSpec((1, 1, BB), lambda i: (i, 0, 0)),
            pl.BlockSpec((NUM_CLASSES, D), lambda i: (0, 0)),
        ],
        out_specs=pl.BlockSpec((1, 1), lambda i: (0, 0)),
        out_shape=jax.ShapeDtypeStruct((1, 1), jnp.float32),
        scratch_shapes=[pltpu.VMEM((KPAD, D), jnp.bfloat16)],
    )(x, labels_i32.reshape(NBLK, 1, BB), centers)
    return loss[0, 0]
